# Initial kernel scaffold; baseline (speedup 1.0000x reference)
import jax
import jax.numpy as jnp
from jax.experimental import pallas as pl


def kernel(rays, mip_levels, texture):
    raise NotImplementedError("write your pallas kernel here")


# speedup vs baseline: 826.6722x; 826.6722x over previous
import jax
import jax.numpy as jnp
from jax import lax
from jax.experimental import pallas as pl
from jax.experimental.pallas import tpu as pltpu
from jax.experimental.pallas import tpu_sc as plsc

RES = 256
MAX_LEVEL = 4
N_LEVELS = 9.0
LEVEL_RES = [RES >> l for l in range(MAX_LEVEL + 1)]
LEVEL_T = [6 * r * r for r in LEVEL_RES]
T_TOTAL = sum(LEVEL_T)
TPAD = 524288

NC, NS, L = 2, 16, 16
NW = NC * NS
CHUNK = 512
G = CHUNK // L
NTAP = 8


def _dmat(R, transposed):
    shape = (R // 2, R) if transposed else (R, R // 2)
    i = lax.broadcasted_iota(jnp.int32, shape, 1 if transposed else 0)
    j = lax.broadcasted_iota(jnp.int32, shape, 0 if transposed else 1)
    diff = (i >> 1) - j
    eq = 1 - jnp.minimum(jnp.abs(diff), 1)
    return 0.5 * eq.astype(jnp.float32)


def _mip_tc_body(tex_ref, o1, o2, o3, o4):
    outs = (o1, o2, o3, o4)
    for f in range(18):
        m = tex_ref[f]
        for lvl in range(4):
            R = RES >> lvl
            d = _dmat(R, False)
            dt = _dmat(R, True)
            t = jax.lax.dot(m, d, precision=lax.Precision.HIGHEST)
            m = jax.lax.dot(dt, t, precision=lax.Precision.HIGHEST)
            outs[lvl][f] = m


def _build_mips_tc(tex18):
    return pl.pallas_call(
        _mip_tc_body,
        out_shape=[jax.ShapeDtypeStruct((18, RES >> l, RES >> l), jnp.float32)
                   for l in range(1, 5)],
    )(tex18)


def _floor_i32(x):
    t = x.astype(jnp.int32)
    tf = t.astype(jnp.float32)
    return jnp.where(tf > x, t - 1, t)


def _level_taps(lv, u, v, face, lw):
    ri = jnp.right_shift(jnp.int32(256), lv)
    rf = ri.astype(jnp.float32)
    rm1 = ri - 1
    x = u * rf - 0.5
    y = v * rf - 0.5
    x0 = _floor_i32(x)
    y0 = _floor_i32(y)
    fx = x - x0.astype(jnp.float32)
    fy = y - y0.astype(jnp.float32)
    x0i = jnp.minimum(jnp.maximum(x0, 0), rm1)
    x1i = jnp.minimum(x0i + 1, rm1)
    y0i = jnp.minimum(jnp.maximum(y0, 0), rm1)
    y1i = jnp.minimum(y0i + 1, rm1)
    off = jnp.int32(524288) - jnp.right_shift(jnp.int32(524288), 2 * lv)
    bface = off + face * ri * ri
    r0 = bface + y0i * ri
    r1 = bface + y1i * ri
    gx = 1.0 - fx
    gy = 1.0 - fy
    idx = (r0 + x0i, r0 + x1i, r1 + x0i, r1 + x1i)
    w = (gx * gy * lw, fx * gy * lw, gx * fy * lw, fx * fy * lw)
    return idx, w


def _sc_body(dx, dy, dz, dm, tRG, tB,
             outR, outG, outB,
             tabRG, tabB,
             set0, set1, oR, oG, oB, semA, semB):
    cid = lax.axis_index("c")
    sid = lax.axis_index("s")
    wid = sid * NC + cid

    SL = TPAD // NS
    so = sid * SL
    pltpu.sync_copy(tRG.at[pl.ds(so, SL)], tabRG.at[pl.ds(so, SL)])
    pltpu.sync_copy(tB.at[pl.ds(so, SL)], tabB.at[pl.ds(so, SL)])
    plsc.subcore_barrier()

    P = dx.shape[0]
    pix_per_w = P // NW
    nchunks = pix_per_w // CHUNK
    w0 = wid * pix_per_w

    def compute(it, bufs):
        bx, by, bz, bm, idxb, wb, rRG, rB = bufs
        base = w0 + it * CHUNK
        pltpu.sync_copy(dx.at[pl.ds(base, CHUNK)], bx)
        pltpu.sync_copy(dy.at[pl.ds(base, CHUNK)], by)
        pltpu.sync_copy(dz.at[pl.ds(base, CHUNK)], bz)
        pltpu.sync_copy(dm.at[pl.ds(base, CHUNK)], bm)

        @plsc.parallel_loop(0, G)
        def grp(g):
            s = g * L
            vx = bx[pl.ds(s, L)]
            vy = by[pl.ds(s, L)]
            vz = bz[pl.ds(s, L)]
            vm = bm[pl.ds(s, L)]
            ax, ay, az = jnp.abs(vx), jnp.abs(vy), jnp.abs(vz)
            is_x = (ax >= ay) & (ax >= az)
            is_y = ((ax < ay) | (ax < az)) & (ay >= az)
            xpos, ypos, zpos = vx > 0, vy > 0, vz > 0
            i0, i1 = jnp.int32(0), jnp.int32(1)
            face = jnp.where(
                is_x, jnp.where(xpos, i0, i1),
                jnp.where(is_y, jnp.where(ypos, i0 + 2, i1 + 2),
                          jnp.where(zpos, i0 + 4, i1 + 4)))
            ma = jnp.where(is_x, ax, jnp.where(is_y, ay, az))
            ma = jnp.maximum(ma, 1e-12)
            sc_ = jnp.where(is_x, jnp.where(xpos, -vz, vz),
                            jnp.where(is_y, vx, jnp.where(zpos, vx, -vx)))
            tc_ = jnp.where(is_y, jnp.where(ypos, vz, -vz), -vy)
            inv = 1.0 / ma
            u = 0.5 * (sc_ * inv + 1.0)
            v = 0.5 * (tc_ * inv + 1.0)

            lvl = jnp.minimum(jnp.maximum(vm * N_LEVELS, 0.0), 4.0)
            l0 = lvl.astype(jnp.int32)
            fr = lvl - l0.astype(jnp.float32)
            l1 = jnp.minimum(l0 + 1, 4)

            idx0, w0_ = _level_taps(l0, u, v, face, 1.0 - fr)
            idx1, w1_ = _level_taps(l1, u, v, face, fr)
            gb = g * (NTAP * L)
            for t in range(4):
                idxb[pl.ds(gb + t * L, L)] = idx0[t]
                wb[pl.ds(gb + t * L, L)] = w0_[t]
                idxb[pl.ds(gb + (t + 4) * L, L)] = idx1[t]
                wb[pl.ds(gb + (t + 4) * L, L)] = w1_[t]

    def fire(bufs, sem):
        _, _, _, _, idxb, _, rRG, rB = bufs
        pltpu.async_copy(tabRG.at[idxb], rRG, sem)
        pltpu.async_copy(tabB.at[idxb], rB, sem)

    def drain(bufs, sem):
        _, _, _, _, idxb, _, rRG, rB = bufs
        pltpu.make_async_copy(tabRG.at[idxb], rRG, sem).wait()
        pltpu.make_async_copy(tabB.at[idxb], rB, sem).wait()

    def flush(it, bufs):
        _, _, _, _, _, wb, rRG, rB = bufs
        base = w0 + it * CHUNK

        @plsc.parallel_loop(0, G)
        def grp2(g):
            s = g * L
            aR = jnp.zeros((L,), jnp.float32)
            aG = jnp.zeros((L,), jnp.float32)
            aB = jnp.zeros((L,), jnp.float32)
            gb = g * (NTAP * L)
            for t in range(NTAP):
                w = wb[pl.ds(gb + t * L, L)]
                rg = rRG[pl.ds(gb + t * L, L)]
                vr = lax.bitcast_convert_type(
                    jnp.bitwise_and(rg, jnp.int32(-65536)), jnp.float32)
                vg = lax.bitcast_convert_type(
                    jnp.left_shift(rg, 16), jnp.float32)
                aR = aR + w * vr
                aG = aG + w * vg
                aB = aB + w * rB[pl.ds(gb + t * L, L)]
            oR[pl.ds(s, L)] = aR
            oG[pl.ds(s, L)] = aG
            oB[pl.ds(s, L)] = aB

        pltpu.sync_copy(oR, outR.at[pl.ds(base, CHUNK)])
        pltpu.sync_copy(oG, outG.at[pl.ds(base, CHUNK)])
        pltpu.sync_copy(oB, outB.at[pl.ds(base, CHUNK)])

    compute(0, set0)
    fire(set0, semA)

    def body(j, carry):
        i0 = 2 * j
        compute(i0 + 1, set1)
        fire(set1, semB)
        drain(set0, semA)
        flush(i0, set0)
        nxt = jnp.minimum(i0 + 2, nchunks - 1)
        compute(nxt, set0)
        fire(set0, semA)
        drain(set1, semB)
        flush(i0 + 1, set1)
        return carry

    lax.fori_loop(0, nchunks // 2, body, 0)
    drain(set0, semA)


def _bufset():
    return (
        pltpu.VMEM((CHUNK,), jnp.float32),
        pltpu.VMEM((CHUNK,), jnp.float32),
        pltpu.VMEM((CHUNK,), jnp.float32),
        pltpu.VMEM((CHUNK,), jnp.float32),
        pltpu.VMEM((CHUNK * NTAP,), jnp.int32),
        pltpu.VMEM((CHUNK * NTAP,), jnp.float32),
        pltpu.VMEM((CHUNK * NTAP,), jnp.int32),
        pltpu.VMEM((CHUNK * NTAP,), jnp.float32),
    )


def _sc_encode(P):
    return pl.kernel(
        _sc_body,
        out_type=[jax.ShapeDtypeStruct((P,), jnp.float32) for _ in range(3)],
        mesh=plsc.VectorSubcoreMesh(core_axis_name="c", subcore_axis_name="s"),
        scratch_types=[
            pltpu.VMEM_SHARED((TPAD,), jnp.int32),
            pltpu.VMEM_SHARED((TPAD,), jnp.float32),
            _bufset(),
            _bufset(),
            pltpu.VMEM((CHUNK,), jnp.float32),
            pltpu.VMEM((CHUNK,), jnp.float32),
            pltpu.VMEM((CHUNK,), jnp.float32),
            pltpu.SemaphoreType.DMA,
            pltpu.SemaphoreType.DMA,
        ],
    )


def kernel(rays, mip_levels, texture):
    N, H, W, _ = rays.shape
    P = N * H * W
    d = rays.reshape(P, 3)
    dx, dy, dz = d[:, 0], d[:, 1], d[:, 2]
    dm = mip_levels.reshape(P)

    tex18 = jnp.transpose(texture[0], (3, 0, 1, 2)).reshape(18, RES, RES)
    m1, m2, m3, m4 = _build_mips_tc(tex18)
    parts = [tex18.reshape(3, LEVEL_T[0]),
             m1.reshape(3, LEVEL_T[1]),
             m2.reshape(3, LEVEL_T[2]),
             m3.reshape(3, LEVEL_T[3]),
             m4.reshape(3, LEVEL_T[4]),
             jnp.zeros((3, TPAD - T_TOTAL), jnp.float32)]
    tab = jnp.concatenate(parts, axis=1)
    rb16 = lax.bitcast_convert_type(tab[0].astype(jnp.bfloat16), jnp.uint16)
    gb16 = lax.bitcast_convert_type(tab[1].astype(jnp.bfloat16), jnp.uint16)
    packed = (rb16.astype(jnp.uint32) << 16) | gb16.astype(jnp.uint32)
    tabRG = lax.bitcast_convert_type(packed, jnp.int32)

    oR, oG, oB = _sc_encode(P)(dx, dy, dz, dm, tabRG, tab[2])
    return jnp.stack([oR, oG, oB]).reshape(3, H, W)
